# baseline (speedup 1.0000x reference)
import functools

import jax
import jax.numpy as jnp
from jax import lax
from jax.experimental import pallas as pl
from jax.experimental.pallas import tpu as pltpu
from jax.experimental.pallas import tpu_sc as plsc

ALPHA = 0.5
N_NODES = 100000
NPAD = 102400
EPAD = 3276800
NC, NS = 2, 16
NW = NC * NS
CHUNK = 2048
ROWS = CHUNK // 128
NCH0, NCH1 = 90, 10
SLICE = NPAD // NS
TC_R, TC_C = NPAD // 128, 128

_mesh = plsc.VectorSubcoreMesh(core_axis_name="c", subcore_axis_name="s")
_params = pltpu.CompilerParams(needs_layout_passes=False)


def _zero_acc_slice(e_v, acc, sid):
    def body(i, _):
        e_v[pl.ds(i * 16, 16)] = jnp.zeros((16,), jnp.float32)
        return 0
    lax.fori_loop(0, 2048 // 16, body, 0)
    src = e_v.at[pl.ds(0, 2048)]
    for k in range(SLICE // 2048):
        pltpu.sync_copy(src, acc.at[pl.ds(sid * SLICE + k * 2048, 2048)])
    rem = SLICE % 2048
    if rem:
        pltpu.sync_copy(e_v.at[pl.ds(0, rem)],
                        acc.at[pl.ds(sid * SLICE + SLICE - rem, rem)])


def _issue_ones_scatter(ones_v, col_v, acc, sem):
    for j in range(ROWS):
        pltpu.async_copy(ones_v, acc.at[col_v.at[j]], sem, add=True)


def _drain_ones_scatter(ones_v, col_v, acc, sem):
    for j in range(ROWS):
        pltpu.make_async_copy(ones_v, acc.at[col_v.at[j]], sem).wait()


def _issue_scatter(vals_v, col_v, acc, sem):
    for j in range(ROWS):
        pltpu.async_copy(vals_v.at[pl.ds(j * 128, 128)], acc.at[col_v.at[j]],
                         sem, add=True)


def _drain_scatter(vals_v, col_v, acc, sem):
    for j in range(ROWS):
        pltpu.make_async_copy(vals_v.at[pl.ds(j * 128, 128)],
                              acc.at[col_v.at[j]], sem).wait()


@functools.partial(
    pl.kernel,
    out_type=jax.ShapeDtypeStruct((NC, 2, NPAD), jnp.float32),
    mesh=_mesh,
    scratch_types=[
        pltpu.VMEM((NPAD,), jnp.float32),
        pltpu.VMEM((CHUNK,), jnp.int32),
        pltpu.VMEM((ROWS, 128), jnp.int32),
        pltpu.VMEM((ROWS, 128), jnp.int32),
        pltpu.VMEM((CHUNK,), jnp.float32),
        pltpu.VMEM((CHUNK,), jnp.float32),
        pltpu.VMEM((128,), jnp.float32),
        pltpu.VMEM_SHARED((NPAD,), jnp.float32),
        pltpu.VMEM_SHARED((NPAD,), jnp.float32),
        pltpu.SemaphoreType.DMA,
        pltpu.SemaphoreType.DMA,
        pltpu.SemaphoreType.DMA,
    ],
    compiler_params=_params,
)
def _fused_kernel(e_hbm, row_hbm, col_hbm, out_hbm,
                  e_v, row_v, col_v0, col_v1, vals_v0, vals_v1, ones_v,
                  acc, dacc, sem_in, s0, s1):
    cid = lax.axis_index("c")
    sid = lax.axis_index("s")

    def fill_ones(i, _):
        ones_v[pl.ds(i * 16, 16)] = jnp.ones((16,), jnp.float32)
        return 0
    lax.fori_loop(0, 128 // 16, fill_ones, 0)

    _zero_acc_slice(e_v, acc, sid)
    _zero_acc_slice(e_v, dacc, sid)
    pltpu.sync_copy(e_hbm, e_v)
    plsc.subcore_barrier()

    ebase = jnp.where(cid == 0, sid * (NCH0 * CHUNK),
                      NS * (NCH0 * CHUNK) + sid * (NCH1 * CHUNK))
    nch_half = jnp.where(cid == 0, NCH0 // 2, NCH1 // 2)
    sems = (s0, s1)
    cols = (col_v0, col_v1)
    vals = (vals_v0, vals_v1)

    def half(ci, b, drain):
        cb = cols[b]
        vb = vals[b]
        if drain:
            _drain_scatter(vb, cb, acc, sems[b])
            _drain_ones_scatter(ones_v, cb, dacc, sems[b])
        e0 = pl.multiple_of(ebase + ci * CHUNK, 128)
        r0 = pl.multiple_of(ebase // 128 + ci * ROWS, 8)
        d1 = pltpu.async_copy(row_hbm.at[pl.ds(e0, CHUNK)], row_v, sem_in)
        d2 = pltpu.async_copy(col_hbm.at[pl.ds(r0, ROWS), :], cb, sem_in)
        d1.wait()
        d2.wait()

        @plsc.parallel_loop(0, CHUNK // 16, unroll=8)
        def gather(j):
            rvec = row_v[pl.ds(j * 16, 16)]
            vb[pl.ds(j * 16, 16)] = plsc.load_gather(e_v, [rvec])

        _issue_scatter(vb, cb, acc, sems[b])
        _issue_ones_scatter(ones_v, cb, dacc, sems[b])

    half(0, 0, False)
    half(1, 1, False)

    def body(k, _):
        half(2 * k, 0, True)
        half(2 * k + 1, 1, True)
        return 0
    lax.fori_loop(1, nch_half, body, 0)
    for b in range(2):
        _drain_scatter(vals[b], cols[b], acc, sems[b])
        _drain_ones_scatter(ones_v, cols[b], dacc, sems[b])

    plsc.subcore_barrier()
    bounce = e_v.at[pl.ds(0, SLICE)]
    pltpu.sync_copy(dacc.at[pl.ds(sid * SLICE, SLICE)], bounce)
    pltpu.sync_copy(bounce, out_hbm.at[cid, 0, pl.ds(sid * SLICE, SLICE)])
    pltpu.sync_copy(acc.at[pl.ds(sid * SLICE, SLICE)], bounce)
    pltpu.sync_copy(bounce, out_hbm.at[cid, 1, pl.ds(sid * SLICE, SLICE)])


@functools.partial(
    pl.kernel,
    out_type=jax.ShapeDtypeStruct((NC, NPAD), jnp.float32),
    mesh=_mesh,
    scratch_types=[
        pltpu.VMEM((NPAD,), jnp.float32),
        pltpu.VMEM((CHUNK,), jnp.int32),
        pltpu.VMEM((ROWS, 128), jnp.int32),
        pltpu.VMEM((ROWS, 128), jnp.int32),
        pltpu.VMEM((CHUNK,), jnp.float32),
        pltpu.VMEM((CHUNK,), jnp.float32),
        pltpu.VMEM_SHARED((NPAD,), jnp.float32),
        pltpu.SemaphoreType.DMA,
        pltpu.SemaphoreType.DMA,
        pltpu.SemaphoreType.DMA,
    ],
    compiler_params=_params,
)
def _scatter_kernel(e_hbm, row_hbm, col_hbm, out_hbm,
                    e_v, row_v, col_v0, col_v1, vals_v0, vals_v1, acc,
                    sem_in, s0, s1):
    cid = lax.axis_index("c")
    sid = lax.axis_index("s")
    _zero_acc_slice(e_v, acc, sid)
    pltpu.sync_copy(e_hbm, e_v)
    plsc.subcore_barrier()

    ebase = jnp.where(cid == 0, sid * (NCH0 * CHUNK),
                      NS * (NCH0 * CHUNK) + sid * (NCH1 * CHUNK))
    nch_half = jnp.where(cid == 0, NCH0 // 2, NCH1 // 2)
    sems = (s0, s1)
    cols = (col_v0, col_v1)
    vals = (vals_v0, vals_v1)

    def half(ci, b, drain):
        cb = cols[b]
        vb = vals[b]
        if drain:
            _drain_scatter(vb, cb, acc, sems[b])
        e0 = pl.multiple_of(ebase + ci * CHUNK, 128)
        r0 = pl.multiple_of(ebase // 128 + ci * ROWS, 8)
        d1 = pltpu.async_copy(row_hbm.at[pl.ds(e0, CHUNK)], row_v, sem_in)
        d2 = pltpu.async_copy(col_hbm.at[pl.ds(r0, ROWS), :], cb, sem_in)
        d1.wait()
        d2.wait()

        @plsc.parallel_loop(0, CHUNK // 16, unroll=8)
        def gather(j):
            rvec = row_v[pl.ds(j * 16, 16)]
            vb[pl.ds(j * 16, 16)] = plsc.load_gather(e_v, [rvec])

        _issue_scatter(vb, cb, acc, sems[b])

    half(0, 0, False)
    half(1, 1, False)

    def body(k, _):
        half(2 * k, 0, True)
        half(2 * k + 1, 1, True)
        return 0
    lax.fori_loop(1, nch_half, body, 0)
    for b in range(2):
        _drain_scatter(vals[b], cols[b], acc, sems[b])

    plsc.subcore_barrier()
    bounce = e_v.at[pl.ds(0, SLICE)]
    pltpu.sync_copy(acc.at[pl.ds(sid * SLICE, SLICE)], bounce)
    pltpu.sync_copy(bounce, out_hbm.at[cid, pl.ds(sid * SLICE, SLICE)])


def _update_tc(e_ref, s_ref, d_ref, out_ref):
    s = s_ref[0] + s_ref[1]
    deg = d_ref[0] + d_ref[1]
    msg = jnp.where(deg > 0.0, s / jnp.where(deg > 0.0, deg, 1.0), 0.0)
    out_ref[...] = e_ref[...] * ALPHA + (1.0 - ALPHA) * msg


_update_call = pl.pallas_call(
    _update_tc, out_shape=jax.ShapeDtypeStruct((TC_R, TC_C), jnp.float32))


def kernel(e, edge_index, prop_layers):
    n = e.shape[0]
    ne = edge_index.shape[1]
    row = edge_index[0].astype(jnp.int32)
    col = edge_index[1].astype(jnp.int32)
    pad_e = EPAD - ne
    fill = jnp.full((pad_e,), n, jnp.int32)
    row_p = jnp.concatenate([row, fill])
    col_r = jnp.concatenate([col, fill]).reshape(EPAD // 128, 128)
    e_p = jnp.concatenate([e, jnp.zeros((NPAD - n,), jnp.float32)])

    ds_parts = _fused_kernel(e_p, row_p, col_r)
    deg_parts = ds_parts[:, 0].reshape(NC, TC_R, TC_C)
    s_parts = ds_parts[:, 1].reshape(NC, TC_R, TC_C)
    e1 = _update_call(e_p.reshape(TC_R, TC_C), s_parts, deg_parts)

    def body(_, e_cur):
        s_p = _scatter_kernel(e_cur, row_p, col_r)
        e_new = _update_call(e_cur.reshape(TC_R, TC_C),
                             s_p.reshape(NC, TC_R, TC_C), deg_parts)
        return e_new.reshape(NPAD)

    e_fin = lax.fori_loop(1, prop_layers, body, e1.reshape(NPAD))
    return e_fin[:n]


# speedup vs baseline: 1.0036x; 1.0036x over previous
import functools

import jax
import jax.numpy as jnp
from jax import lax
from jax.experimental import pallas as pl
from jax.experimental.pallas import tpu as pltpu
from jax.experimental.pallas import tpu_sc as plsc

ALPHA = 0.5
N_NODES = 100000
NPAD = 102400
EPAD = 3276800
NC, NS = 2, 16
NW = NC * NS
CHUNK = 2048
ROWS = CHUNK // 128
NCH0, NCH1 = 80, 20
SLICE = NPAD // NS
TC_R, TC_C = NPAD // 128, 128

_mesh = plsc.VectorSubcoreMesh(core_axis_name="c", subcore_axis_name="s")
_params = pltpu.CompilerParams(needs_layout_passes=False)


def _zero_acc_slice(e_v, acc, sid):
    def body(i, _):
        e_v[pl.ds(i * 16, 16)] = jnp.zeros((16,), jnp.float32)
        return 0
    lax.fori_loop(0, 2048 // 16, body, 0)
    src = e_v.at[pl.ds(0, 2048)]
    for k in range(SLICE // 2048):
        pltpu.sync_copy(src, acc.at[pl.ds(sid * SLICE + k * 2048, 2048)])
    rem = SLICE % 2048
    if rem:
        pltpu.sync_copy(e_v.at[pl.ds(0, rem)],
                        acc.at[pl.ds(sid * SLICE + SLICE - rem, rem)])


def _issue_ones_scatter(ones_v, col_v, acc, sem):
    for j in range(ROWS):
        pltpu.async_copy(ones_v, acc.at[col_v.at[j]], sem, add=True)


def _drain_ones_scatter(ones_v, col_v, acc, sem):
    for j in range(ROWS):
        pltpu.make_async_copy(ones_v, acc.at[col_v.at[j]], sem).wait()


def _issue_scatter(vals_v, col_v, acc, sem):
    for j in range(ROWS):
        pltpu.async_copy(vals_v.at[pl.ds(j * 128, 128)], acc.at[col_v.at[j]],
                         sem, add=True)


def _drain_scatter(vals_v, col_v, acc, sem):
    for j in range(ROWS):
        pltpu.make_async_copy(vals_v.at[pl.ds(j * 128, 128)],
                              acc.at[col_v.at[j]], sem).wait()


@functools.partial(
    pl.kernel,
    out_type=jax.ShapeDtypeStruct((NC, 2, NPAD), jnp.float32),
    mesh=_mesh,
    scratch_types=[
        pltpu.VMEM((NPAD,), jnp.float32),
        pltpu.VMEM((CHUNK,), jnp.int32),
        pltpu.VMEM((ROWS, 128), jnp.int32),
        pltpu.VMEM((ROWS, 128), jnp.int32),
        pltpu.VMEM((CHUNK,), jnp.float32),
        pltpu.VMEM((CHUNK,), jnp.float32),
        pltpu.VMEM((128,), jnp.float32),
        pltpu.VMEM_SHARED((NPAD,), jnp.float32),
        pltpu.VMEM_SHARED((NPAD,), jnp.float32),
        pltpu.SemaphoreType.DMA,
        pltpu.SemaphoreType.DMA,
        pltpu.SemaphoreType.DMA,
    ],
    compiler_params=_params,
)
def _fused_kernel(e_hbm, row_hbm, col_hbm, out_hbm,
                  e_v, row_v, col_v0, col_v1, vals_v0, vals_v1, ones_v,
                  acc, dacc, sem_in, s0, s1):
    cid = lax.axis_index("c")
    sid = lax.axis_index("s")

    def fill_ones(i, _):
        ones_v[pl.ds(i * 16, 16)] = jnp.ones((16,), jnp.float32)
        return 0
    lax.fori_loop(0, 128 // 16, fill_ones, 0)

    _zero_acc_slice(e_v, acc, sid)
    _zero_acc_slice(e_v, dacc, sid)
    pltpu.sync_copy(e_hbm, e_v)
    plsc.subcore_barrier()

    ebase = jnp.where(cid == 0, sid * (NCH0 * CHUNK),
                      NS * (NCH0 * CHUNK) + sid * (NCH1 * CHUNK))
    nch_half = jnp.where(cid == 0, NCH0 // 2, NCH1 // 2)
    sems = (s0, s1)
    cols = (col_v0, col_v1)
    vals = (vals_v0, vals_v1)

    def half(ci, b, drain):
        cb = cols[b]
        vb = vals[b]
        if drain:
            _drain_scatter(vb, cb, acc, sems[b])
            _drain_ones_scatter(ones_v, cb, dacc, sems[b])
        e0 = pl.multiple_of(ebase + ci * CHUNK, 128)
        r0 = pl.multiple_of(ebase // 128 + ci * ROWS, 8)
        d1 = pltpu.async_copy(row_hbm.at[pl.ds(e0, CHUNK)], row_v, sem_in)
        d2 = pltpu.async_copy(col_hbm.at[pl.ds(r0, ROWS), :], cb, sem_in)
        d1.wait()
        d2.wait()

        @plsc.parallel_loop(0, CHUNK // 16, unroll=8)
        def gather(j):
            rvec = row_v[pl.ds(j * 16, 16)]
            vb[pl.ds(j * 16, 16)] = plsc.load_gather(e_v, [rvec])

        _issue_scatter(vb, cb, acc, sems[b])
        _issue_ones_scatter(ones_v, cb, dacc, sems[b])

    half(0, 0, False)
    half(1, 1, False)

    def body(k, _):
        half(2 * k, 0, True)
        half(2 * k + 1, 1, True)
        return 0
    lax.fori_loop(1, nch_half, body, 0)
    for b in range(2):
        _drain_scatter(vals[b], cols[b], acc, sems[b])
        _drain_ones_scatter(ones_v, cols[b], dacc, sems[b])

    plsc.subcore_barrier()
    bounce = e_v.at[pl.ds(0, SLICE)]
    pltpu.sync_copy(dacc.at[pl.ds(sid * SLICE, SLICE)], bounce)
    pltpu.sync_copy(bounce, out_hbm.at[cid, 0, pl.ds(sid * SLICE, SLICE)])
    pltpu.sync_copy(acc.at[pl.ds(sid * SLICE, SLICE)], bounce)
    pltpu.sync_copy(bounce, out_hbm.at[cid, 1, pl.ds(sid * SLICE, SLICE)])


@functools.partial(
    pl.kernel,
    out_type=jax.ShapeDtypeStruct((NC, NPAD), jnp.float32),
    mesh=_mesh,
    scratch_types=[
        pltpu.VMEM((NPAD,), jnp.float32),
        pltpu.VMEM((CHUNK,), jnp.int32),
        pltpu.VMEM((ROWS, 128), jnp.int32),
        pltpu.VMEM((ROWS, 128), jnp.int32),
        pltpu.VMEM((CHUNK,), jnp.float32),
        pltpu.VMEM((CHUNK,), jnp.float32),
        pltpu.VMEM_SHARED((NPAD,), jnp.float32),
        pltpu.SemaphoreType.DMA,
        pltpu.SemaphoreType.DMA,
        pltpu.SemaphoreType.DMA,
    ],
    compiler_params=_params,
)
def _scatter_kernel(e_hbm, row_hbm, col_hbm, out_hbm,
                    e_v, row_v, col_v0, col_v1, vals_v0, vals_v1, acc,
                    sem_in, s0, s1):
    cid = lax.axis_index("c")
    sid = lax.axis_index("s")
    _zero_acc_slice(e_v, acc, sid)
    pltpu.sync_copy(e_hbm, e_v)
    plsc.subcore_barrier()

    ebase = jnp.where(cid == 0, sid * (NCH0 * CHUNK),
                      NS * (NCH0 * CHUNK) + sid * (NCH1 * CHUNK))
    nch_half = jnp.where(cid == 0, NCH0 // 2, NCH1 // 2)
    sems = (s0, s1)
    cols = (col_v0, col_v1)
    vals = (vals_v0, vals_v1)

    def half(ci, b, drain):
        cb = cols[b]
        vb = vals[b]
        if drain:
            _drain_scatter(vb, cb, acc, sems[b])
        e0 = pl.multiple_of(ebase + ci * CHUNK, 128)
        r0 = pl.multiple_of(ebase // 128 + ci * ROWS, 8)
        d1 = pltpu.async_copy(row_hbm.at[pl.ds(e0, CHUNK)], row_v, sem_in)
        d2 = pltpu.async_copy(col_hbm.at[pl.ds(r0, ROWS), :], cb, sem_in)
        d1.wait()
        d2.wait()

        @plsc.parallel_loop(0, CHUNK // 16, unroll=8)
        def gather(j):
            rvec = row_v[pl.ds(j * 16, 16)]
            vb[pl.ds(j * 16, 16)] = plsc.load_gather(e_v, [rvec])

        _issue_scatter(vb, cb, acc, sems[b])

    half(0, 0, False)
    half(1, 1, False)

    def body(k, _):
        half(2 * k, 0, True)
        half(2 * k + 1, 1, True)
        return 0
    lax.fori_loop(1, nch_half, body, 0)
    for b in range(2):
        _drain_scatter(vals[b], cols[b], acc, sems[b])

    plsc.subcore_barrier()
    bounce = e_v.at[pl.ds(0, SLICE)]
    pltpu.sync_copy(acc.at[pl.ds(sid * SLICE, SLICE)], bounce)
    pltpu.sync_copy(bounce, out_hbm.at[cid, pl.ds(sid * SLICE, SLICE)])


def _update_tc(e_ref, s_ref, d_ref, out_ref):
    s = s_ref[0] + s_ref[1]
    deg = d_ref[0] + d_ref[1]
    msg = jnp.where(deg > 0.0, s / jnp.where(deg > 0.0, deg, 1.0), 0.0)
    out_ref[...] = e_ref[...] * ALPHA + (1.0 - ALPHA) * msg


_update_call = pl.pallas_call(
    _update_tc, out_shape=jax.ShapeDtypeStruct((TC_R, TC_C), jnp.float32))


def kernel(e, edge_index, prop_layers):
    n = e.shape[0]
    ne = edge_index.shape[1]
    row = edge_index[0].astype(jnp.int32)
    col = edge_index[1].astype(jnp.int32)
    pad_e = EPAD - ne
    fill = jnp.full((pad_e,), n, jnp.int32)
    row_p = jnp.concatenate([row, fill])
    col_r = jnp.concatenate([col, fill]).reshape(EPAD // 128, 128)
    e_p = jnp.concatenate([e, jnp.zeros((NPAD - n,), jnp.float32)])

    ds_parts = _fused_kernel(e_p, row_p, col_r)
    deg_parts = ds_parts[:, 0].reshape(NC, TC_R, TC_C)
    s_parts = ds_parts[:, 1].reshape(NC, TC_R, TC_C)
    e1 = _update_call(e_p.reshape(TC_R, TC_C), s_parts, deg_parts)

    def body(_, e_cur):
        s_p = _scatter_kernel(e_cur, row_p, col_r)
        e_new = _update_call(e_cur.reshape(TC_R, TC_C),
                             s_p.reshape(NC, TC_R, TC_C), deg_parts)
        return e_new.reshape(NPAD)

    e_fin = lax.fori_loop(1, prop_layers, body, e1.reshape(NPAD))
    return e_fin[:n]


# speedup vs baseline: 1.0548x; 1.0510x over previous
import functools

import jax
import jax.numpy as jnp
from jax import lax
from jax.experimental import pallas as pl
from jax.experimental.pallas import tpu as pltpu
from jax.experimental.pallas import tpu_sc as plsc

ALPHA = 0.5
N_NODES = 100000
NPAD = 102400
MAIN_ROWS = 25000
PAD_ROWS = 600
EPAD = 3276800
NC, NS = 2, 16
NW = NC * NS
CHUNK = 2048
ROWS = CHUNK // 128
NCH0, NCH1 = 80, 20
SLICE = NPAD // NS
TC_R, TC_C = NPAD // 128, 128

_mesh = plsc.VectorSubcoreMesh(core_axis_name="c", subcore_axis_name="s")
_params = pltpu.CompilerParams(needs_layout_passes=False)


def _zero_acc_slice(e_v, acc, sid):
    def body(i, _):
        e_v[pl.ds(i * 16, 16)] = jnp.zeros((16,), jnp.float32)
        return 0
    lax.fori_loop(0, 2048 // 16, body, 0)
    src = e_v.at[pl.ds(0, 2048)]
    for k in range(SLICE // 2048):
        pltpu.sync_copy(src, acc.at[pl.ds(sid * SLICE + k * 2048, 2048)])
    rem = SLICE % 2048
    if rem:
        pltpu.sync_copy(e_v.at[pl.ds(0, rem)],
                        acc.at[pl.ds(sid * SLICE + SLICE - rem, rem)])


def _issue_ones_scatter(ones_v, col_v, acc, sem):
    for j in range(ROWS):
        pltpu.async_copy(ones_v, acc.at[col_v.at[j]], sem, add=True)


def _drain_ones_scatter(ones_v, col_v, acc, sem):
    for j in range(ROWS):
        pltpu.make_async_copy(ones_v, acc.at[col_v.at[j]], sem).wait()


def _issue_scatter(vals_v, col_v, acc, sem):
    for j in range(ROWS):
        pltpu.async_copy(vals_v.at[pl.ds(j * 128, 128)], acc.at[col_v.at[j]],
                         sem, add=True)


def _drain_scatter(vals_v, col_v, acc, sem):
    for j in range(ROWS):
        pltpu.make_async_copy(vals_v.at[pl.ds(j * 128, 128)],
                              acc.at[col_v.at[j]], sem).wait()


@functools.partial(
    pl.kernel,
    out_type=jax.ShapeDtypeStruct((NC, 2, NPAD), jnp.float32),
    mesh=_mesh,
    scratch_types=[
        pltpu.VMEM((NPAD,), jnp.float32),
        pltpu.VMEM((ROWS, 128), jnp.int32),
        pltpu.VMEM((ROWS, 128), jnp.int32),
        pltpu.VMEM((ROWS, 128), jnp.int32),
        pltpu.VMEM((CHUNK,), jnp.float32),
        pltpu.VMEM((CHUNK,), jnp.float32),
        pltpu.VMEM((128,), jnp.float32),
        pltpu.VMEM_SHARED((NPAD,), jnp.float32),
        pltpu.VMEM_SHARED((NPAD,), jnp.float32),
        pltpu.SemaphoreType.DMA,
        pltpu.SemaphoreType.DMA,
        pltpu.SemaphoreType.DMA,
    ],
    compiler_params=_params,
)
def _fused_kernel(e_hbm, ei_hbm, pad_hbm, out_hbm,
                  e_v, row_v, col_v0, col_v1, vals_v0, vals_v1, ones_v,
                  acc, dacc, sem_in, s0, s1):
    cid = lax.axis_index("c")
    sid = lax.axis_index("s")

    def fill_ones(i, _):
        ones_v[pl.ds(i * 16, 16)] = jnp.ones((16,), jnp.float32)
        return 0
    lax.fori_loop(0, 128 // 16, fill_ones, 0)

    _zero_acc_slice(e_v, acc, sid)
    _zero_acc_slice(e_v, dacc, sid)

    def ztail(i, _):
        e_v[pl.ds(N_NODES + i * 16, 16)] = jnp.zeros((16,), jnp.float32)
        return 0
    lax.fori_loop(0, (NPAD - N_NODES) // 16, ztail, 0)
    pltpu.sync_copy(e_hbm, e_v.at[pl.ds(0, N_NODES)])
    plsc.subcore_barrier()

    rbase = jnp.where(cid == 0, sid * (NCH0 * ROWS),
                      NS * (NCH0 * ROWS) + sid * (NCH1 * ROWS))
    nch_half = jnp.where(cid == 0, NCH0 // 2, NCH1 // 2)
    sems = (s0, s1)
    cols = (col_v0, col_v1)
    vals = (vals_v0, vals_v1)

    def half(ci, b, drain):
        cb = cols[b]
        vb = vals[b]
        if drain:
            _drain_scatter(vb, cb, acc, sems[b])
            _drain_ones_scatter(ones_v, cb, dacc, sems[b])
        r0 = pl.multiple_of(rbase + ci * ROWS, 8)

        @pl.when(r0 + ROWS <= MAIN_ROWS)
        def _():
            d1 = pltpu.async_copy(ei_hbm.at[0, pl.ds(r0, ROWS), :], row_v,
                                  sem_in)
            d2 = pltpu.async_copy(ei_hbm.at[1, pl.ds(r0, ROWS), :], cb,
                                  sem_in)
            d1.wait()
            d2.wait()

        @pl.when(r0 >= MAIN_ROWS)
        def _():
            rp = pl.multiple_of(r0 - MAIN_ROWS, 8)
            d1 = pltpu.async_copy(pad_hbm.at[pl.ds(rp, ROWS), :], row_v,
                                  sem_in)
            d2 = pltpu.async_copy(pad_hbm.at[pl.ds(rp, ROWS), :], cb, sem_in)
            d1.wait()
            d2.wait()

        @pl.when((r0 < MAIN_ROWS) & (r0 + ROWS > MAIN_ROWS))
        def _():
            h = ROWS // 2
            d1 = pltpu.async_copy(ei_hbm.at[0, pl.ds(r0, h), :],
                                  row_v.at[pl.ds(0, h), :], sem_in)
            d2 = pltpu.async_copy(ei_hbm.at[1, pl.ds(r0, h), :],
                                  cb.at[pl.ds(0, h), :], sem_in)
            d3 = pltpu.async_copy(pad_hbm.at[pl.ds(0, h), :],
                                  row_v.at[pl.ds(h, h), :], sem_in)
            d4 = pltpu.async_copy(pad_hbm.at[pl.ds(0, h), :],
                                  cb.at[pl.ds(h, h), :], sem_in)
            d1.wait()
            d2.wait()
            d3.wait()
            d4.wait()

        @plsc.parallel_loop(0, ROWS, unroll=2)
        def gather(r):
            for k in range(8):
                rvec = row_v[r, pl.ds(k * 16, 16)]
                vb[pl.ds(r * 128 + k * 16, 16)] = plsc.load_gather(
                    e_v, [rvec])

        _issue_scatter(vb, cb, acc, sems[b])
        _issue_ones_scatter(ones_v, cb, dacc, sems[b])

    half(0, 0, False)
    half(1, 1, False)

    def body(k, _):
        half(2 * k, 0, True)
        half(2 * k + 1, 1, True)
        return 0
    lax.fori_loop(1, nch_half, body, 0)
    for b in range(2):
        _drain_scatter(vals[b], cols[b], acc, sems[b])
        _drain_ones_scatter(ones_v, cols[b], dacc, sems[b])

    plsc.subcore_barrier()
    bounce = e_v.at[pl.ds(0, SLICE)]
    pltpu.sync_copy(dacc.at[pl.ds(sid * SLICE, SLICE)], bounce)
    pltpu.sync_copy(bounce, out_hbm.at[cid, 0, pl.ds(sid * SLICE, SLICE)])
    pltpu.sync_copy(acc.at[pl.ds(sid * SLICE, SLICE)], bounce)
    pltpu.sync_copy(bounce, out_hbm.at[cid, 1, pl.ds(sid * SLICE, SLICE)])


@functools.partial(
    pl.kernel,
    out_type=jax.ShapeDtypeStruct((NC, NPAD), jnp.float32),
    mesh=_mesh,
    scratch_types=[
        pltpu.VMEM((NPAD,), jnp.float32),
        pltpu.VMEM((ROWS, 128), jnp.int32),
        pltpu.VMEM((ROWS, 128), jnp.int32),
        pltpu.VMEM((ROWS, 128), jnp.int32),
        pltpu.VMEM((CHUNK,), jnp.float32),
        pltpu.VMEM((CHUNK,), jnp.float32),
        pltpu.VMEM_SHARED((NPAD,), jnp.float32),
        pltpu.SemaphoreType.DMA,
        pltpu.SemaphoreType.DMA,
        pltpu.SemaphoreType.DMA,
    ],
    compiler_params=_params,
)
def _scatter_kernel(e_hbm, ei_hbm, pad_hbm, out_hbm,
                     e_v, row_v, col_v0, col_v1, vals_v0, vals_v1, acc,
                     sem_in, s0, s1):
    cid = lax.axis_index("c")
    sid = lax.axis_index("s")
    _zero_acc_slice(e_v, acc, sid)
    pltpu.sync_copy(e_hbm, e_v)
    plsc.subcore_barrier()

    rbase = jnp.where(cid == 0, sid * (NCH0 * ROWS),
                      NS * (NCH0 * ROWS) + sid * (NCH1 * ROWS))
    nch_half = jnp.where(cid == 0, NCH0 // 2, NCH1 // 2)
    sems = (s0, s1)
    cols = (col_v0, col_v1)
    vals = (vals_v0, vals_v1)

    def half(ci, b, drain):
        cb = cols[b]
        vb = vals[b]
        if drain:
            _drain_scatter(vb, cb, acc, sems[b])
        r0 = pl.multiple_of(rbase + ci * ROWS, 8)

        @pl.when(r0 + ROWS <= MAIN_ROWS)
        def _():
            d1 = pltpu.async_copy(ei_hbm.at[0, pl.ds(r0, ROWS), :], row_v,
                                  sem_in)
            d2 = pltpu.async_copy(ei_hbm.at[1, pl.ds(r0, ROWS), :], cb,
                                  sem_in)
            d1.wait()
            d2.wait()

        @pl.when(r0 >= MAIN_ROWS)
        def _():
            rp = pl.multiple_of(r0 - MAIN_ROWS, 8)
            d1 = pltpu.async_copy(pad_hbm.at[pl.ds(rp, ROWS), :], row_v,
                                  sem_in)
            d2 = pltpu.async_copy(pad_hbm.at[pl.ds(rp, ROWS), :], cb, sem_in)
            d1.wait()
            d2.wait()

        @pl.when((r0 < MAIN_ROWS) & (r0 + ROWS > MAIN_ROWS))
        def _():
            h = ROWS // 2
            d1 = pltpu.async_copy(ei_hbm.at[0, pl.ds(r0, h), :],
                                  row_v.at[pl.ds(0, h), :], sem_in)
            d2 = pltpu.async_copy(ei_hbm.at[1, pl.ds(r0, h), :],
                                  cb.at[pl.ds(0, h), :], sem_in)
            d3 = pltpu.async_copy(pad_hbm.at[pl.ds(0, h), :],
                                  row_v.at[pl.ds(h, h), :], sem_in)
            d4 = pltpu.async_copy(pad_hbm.at[pl.ds(0, h), :],
                                  cb.at[pl.ds(h, h), :], sem_in)
            d1.wait()
            d2.wait()
            d3.wait()
            d4.wait()

        @plsc.parallel_loop(0, ROWS, unroll=2)
        def gather(r):
            for k in range(8):
                rvec = row_v[r, pl.ds(k * 16, 16)]
                vb[pl.ds(r * 128 + k * 16, 16)] = plsc.load_gather(
                    e_v, [rvec])

        _issue_scatter(vb, cb, acc, sems[b])

    half(0, 0, False)
    half(1, 1, False)

    def body(k, _):
        half(2 * k, 0, True)
        half(2 * k + 1, 1, True)
        return 0
    lax.fori_loop(1, nch_half, body, 0)
    for b in range(2):
        _drain_scatter(vals[b], cols[b], acc, sems[b])

    plsc.subcore_barrier()
    bounce = e_v.at[pl.ds(0, SLICE)]
    pltpu.sync_copy(acc.at[pl.ds(sid * SLICE, SLICE)], bounce)
    pltpu.sync_copy(bounce, out_hbm.at[cid, pl.ds(sid * SLICE, SLICE)])


def _update_tc(e_ref, s_ref, d_ref, out_ref):
    s = s_ref[0] + s_ref[1]
    deg = d_ref[0] + d_ref[1]
    msg = jnp.where(deg > 0.0, s / jnp.where(deg > 0.0, deg, 1.0), 0.0)
    out_ref[...] = e_ref[...] * ALPHA + (1.0 - ALPHA) * msg


_update_call = pl.pallas_call(
    _update_tc, out_shape=jax.ShapeDtypeStruct((TC_R, TC_C), jnp.float32))


def kernel(e, edge_index, prop_layers):
    n = e.shape[0]
    ei3 = edge_index.astype(jnp.int32).reshape(2, MAIN_ROWS, 128)
    padr = jnp.full((PAD_ROWS, 128), n, jnp.int32)

    ds_parts = _fused_kernel(e, ei3, padr)
    deg_parts = ds_parts[:, 0].reshape(NC, TC_R, TC_C)
    s_parts = ds_parts[:, 1].reshape(NC, TC_R, TC_C)
    e_p = jnp.concatenate([e, jnp.zeros((NPAD - n,), jnp.float32)])
    e1 = _update_call(e_p.reshape(TC_R, TC_C), s_parts, deg_parts)

    def body(_, e_cur):
        s_p = _scatter_kernel(e_cur, ei3, padr)
        e_new = _update_call(e_cur.reshape(TC_R, TC_C),
                             s_p.reshape(NC, TC_R, TC_C), deg_parts)
        return e_new.reshape(NPAD)

    e_fin = lax.fori_loop(1, prop_layers, body, e1.reshape(NPAD))
    return e_fin[:n]
